# baseline (speedup 1.0000x reference)
import functools

import jax
import jax.numpy as jnp
from jax import lax
from jax.experimental import pallas as pl
from jax.experimental.pallas import tpu as pltpu
from jax.experimental.pallas import tpu_sc as plsc

_NUM_CORES = 2
_NUM_SUBCORES = 16
_LANES = 16
_TC_LANES = 128


def _build_a_call(q_len, v_len, max_len, d):
    p = (max_len - 1) // 2
    off = (q_len - 1) - p
    a_rows = q_len + v_len
    nw = _NUM_CORES * _NUM_SUBCORES
    assert a_rows % nw == 0
    bpw = a_rows // nw

    mesh = plsc.VectorSubcoreMesh(core_axis_name="c", subcore_axis_name="s")

    @functools.partial(
        pl.kernel,
        out_type=jax.ShapeDtypeStruct((a_rows * d,), jnp.float32),
        mesh=mesh,
        compiler_params=pltpu.CompilerParams(use_tc_tiling_on_sc=False),
        scratch_types=[
            pltpu.VMEM((max_len * d,), jnp.float32),
            pltpu.VMEM((bpw * d,), jnp.float32),
            pltpu.SemaphoreType.DMA,
        ],
    )
    def build_a(emb_hbm, a_hbm, emb_v, build_v, sem):
        wid = lax.axis_index("c") * _NUM_SUBCORES + lax.axis_index("s")
        pltpu.async_copy(emb_hbm, emb_v, sem).wait()
        bias = wid * bpw - off

        @pl.loop(0, bpw, step=4)
        def _(t):
            for u in range(4):
                k = jnp.minimum(jnp.maximum(bias + (t + u), 0), max_len - 1)
                for h in range(d // _LANES):
                    build_v[pl.ds((t + u) * d + h * _LANES, _LANES)] = (
                        emb_v[pl.ds(k * d + h * _LANES, _LANES)]
                    )

        pltpu.async_copy(build_v, a_hbm.at[pl.ds(wid * bpw * d, bpw * d)], sem).wait()

    return build_a


def _emit_rows_tc(q_len, v_len, max_len, d):
    a_rows = q_len + v_len
    rpb = 8
    assert q_len % rpb == 0

    def body(a_ref, o_ref):
        g = pl.program_id(0)
        for r in range(rpb):
            i = g * rpb + r
            o_ref[r] = a_ref[pl.ds((q_len - 1) - i, v_len), :]

    return pl.pallas_call(
        body,
        grid=(q_len // rpb,),
        in_specs=[
            pl.BlockSpec((a_rows, d), lambda g: (0, 0)),
        ],
        out_specs=pl.BlockSpec((rpb, v_len, d), lambda g: (g, 0, 0)),
        out_shape=jax.ShapeDtypeStruct((q_len, v_len, d), jnp.float32),
    )


def kernel(q, v, embeddings):
    q_len = int(q.shape[1])
    v_len = int(v.shape[1])
    max_len, d = int(embeddings.shape[0]), int(embeddings.shape[1])
    a_flat = _build_a_call(q_len, v_len, max_len, d)(embeddings.reshape(max_len * d))
    a2d = a_flat.reshape(q_len + v_len, d)
    return _emit_rows_tc(q_len, v_len, max_len, d)(a2d)


# speedup vs baseline: 6.5100x; 6.5100x over previous
import dataclasses
import functools

import jax
import jax.numpy as jnp
from jax import lax
from jax.experimental import pallas as pl
from jax.experimental.pallas import tpu as pltpu
from jax.experimental.pallas import tpu_sc as plsc

_NUM_CORES = 2
_NUM_SUBCORES = 16
_LANES = 16


def _build_at_call(q_len, v_len, max_len, d):
    p = (max_len - 1) // 2
    off = (q_len - 1) - p
    a_cols = q_len + v_len
    nw = _NUM_CORES * _NUM_SUBCORES
    assert d == nw
    assert a_cols % _LANES == 0

    mesh = plsc.VectorSubcoreMesh(core_axis_name="c", subcore_axis_name="s")

    cp = pltpu.CompilerParams(use_tc_tiling_on_sc=False)
    if "needs_layout_passes" in pltpu.CompilerParams.__dataclass_fields__:
        cp = dataclasses.replace(cp, needs_layout_passes=False)

    @functools.partial(
        pl.kernel,
        out_type=jax.ShapeDtypeStruct((d * a_cols,), jnp.float32),
        mesh=mesh,
        compiler_params=cp,
        scratch_types=[
            pltpu.VMEM((max_len * d,), jnp.float32),
            pltpu.VMEM((a_cols,), jnp.float32),
            pltpu.SemaphoreType.DMA,
        ],
    )
    def build_at(emb_hbm, at_hbm, emb_v, row_v, sem):
        e = lax.axis_index("c") * _NUM_SUBCORES + lax.axis_index("s")
        pltpu.async_copy(emb_hbm, emb_v, sem).wait()
        lane = lax.iota(jnp.int32, _LANES)

        @pl.loop(0, a_cols, step=_LANES)
        def _(k0):
            kk = jnp.minimum(jnp.maximum((k0 - off) + lane, 0), max_len - 1)
            row_v[pl.ds(k0, _LANES)] = plsc.load_gather(emb_v, [kk * d + e])

        pltpu.async_copy(row_v, at_hbm.at[pl.ds(e * a_cols, a_cols)], sem).wait()

    return build_at


def _emit_rows_tc(q_len, v_len, max_len, d):
    a_cols = q_len + v_len
    rpb = 8
    assert q_len % rpb == 0

    sup = v_len + 128

    def body(at_ref, o_ref):
        g = pl.program_id(0)
        for r in range(rpb):
            i = g * rpb + r
            s = (q_len - 1) - i
            base = pl.multiple_of((s // 128) * 128, 128)
            t = s - base
            win = at_ref[:, pl.ds(base, sup)]
            o_ref[r] = pltpu.roll(win, sup - t, axis=1)[:, :v_len]

    return pl.pallas_call(
        body,
        grid=(q_len // rpb,),
        in_specs=[
            pl.BlockSpec((d, a_cols), lambda g: (0, 0)),
        ],
        out_specs=pl.BlockSpec((rpb, d, v_len), lambda g: (g, 0, 0)),
        out_shape=jax.ShapeDtypeStruct((q_len, d, v_len), jnp.float32),
    )


def kernel(q, v, embeddings):
    q_len = int(q.shape[1])
    v_len = int(v.shape[1])
    max_len, d = int(embeddings.shape[0]), int(embeddings.shape[1])
    at_flat = _build_at_call(q_len, v_len, max_len, d)(embeddings.reshape(max_len * d))
    a_t = at_flat.reshape(d, q_len + v_len)
    out_t = _emit_rows_tc(q_len, v_len, max_len, d)(a_t)
    return jnp.swapaxes(out_t, 1, 2)


# speedup vs baseline: 7.7802x; 1.1951x over previous
import dataclasses
import functools

import jax
import jax.numpy as jnp
from jax import lax
from jax.experimental import pallas as pl
from jax.experimental.pallas import tpu as pltpu
from jax.experimental.pallas import tpu_sc as plsc

_NUM_CORES = 2
_NUM_SUBCORES = 16
_LANES = 16


def _build_at_call(q_len, v_len, max_len, d):
    p = (max_len - 1) // 2
    off = (q_len - 1) - p
    a_cols = q_len + v_len
    nw = _NUM_CORES * _NUM_SUBCORES
    assert d == nw
    assert a_cols % _LANES == 0

    mesh = plsc.VectorSubcoreMesh(core_axis_name="c", subcore_axis_name="s")

    cp = pltpu.CompilerParams(use_tc_tiling_on_sc=False)
    if "needs_layout_passes" in pltpu.CompilerParams.__dataclass_fields__:
        cp = dataclasses.replace(cp, needs_layout_passes=False)

    @functools.partial(
        pl.kernel,
        out_type=jax.ShapeDtypeStruct((d * a_cols,), jnp.float32),
        mesh=mesh,
        compiler_params=cp,
        scratch_types=[
            pltpu.VMEM((max_len * d,), jnp.float32),
            pltpu.VMEM((a_cols,), jnp.float32),
            pltpu.SemaphoreType.DMA,
        ],
    )
    def build_at(emb_hbm, at_hbm, emb_v, row_v, sem):
        e = lax.axis_index("c") * _NUM_SUBCORES + lax.axis_index("s")
        pltpu.async_copy(emb_hbm, emb_v, sem).wait()
        lane = lax.iota(jnp.int32, _LANES)

        @pl.loop(0, a_cols, step=_LANES)
        def _(k0):
            kk = jnp.minimum(jnp.maximum((k0 - off) + lane, 0), max_len - 1)
            row_v[pl.ds(k0, _LANES)] = plsc.load_gather(emb_v, [kk * d + e])

        pltpu.async_copy(row_v, at_hbm.at[pl.ds(e * a_cols, a_cols)], sem).wait()

    return build_at


def _emit_rows_tc(q_len, v_len, max_len, d):
    a_cols = q_len + v_len
    rpb = 16
    assert q_len % rpb == 0

    sup = v_len + 128

    def body(at_ref, o_ref):
        g = pl.program_id(0)
        for r in range(rpb):
            i = g * rpb + r
            s = (q_len - 1) - i
            base = pl.multiple_of((s // 128) * 128, 128)
            t = s - base
            win = at_ref[:, pl.ds(base, sup)]
            o_ref[r] = pltpu.roll(win, sup - t, axis=1)[:, :v_len]

    return pl.pallas_call(
        body,
        grid=(q_len // rpb,),
        in_specs=[
            pl.BlockSpec((d, a_cols), lambda g: (0, 0)),
        ],
        out_specs=pl.BlockSpec((rpb, d, v_len), lambda g: (g, 0, 0)),
        out_shape=jax.ShapeDtypeStruct((q_len, d, v_len), jnp.float32),
    )


def kernel(q, v, embeddings):
    q_len = int(q.shape[1])
    v_len = int(v.shape[1])
    max_len, d = int(embeddings.shape[0]), int(embeddings.shape[1])
    at_flat = _build_at_call(q_len, v_len, max_len, d)(embeddings.reshape(max_len * d))
    a_t = at_flat.reshape(d, q_len + v_len)
    out_t = _emit_rows_tc(q_len, v_len, max_len, d)(a_t)
    return jnp.swapaxes(out_t, 1, 2)


# speedup vs baseline: 8.6767x; 1.1152x over previous
import dataclasses
import functools

import jax
import jax.numpy as jnp
from jax import lax
from jax.experimental import pallas as pl
from jax.experimental.pallas import tpu as pltpu
from jax.experimental.pallas import tpu_sc as plsc

_NUM_CORES = 2
_NUM_SUBCORES = 16
_LANES = 16


def _build_at_call(q_len, v_len, max_len, d):
    p = (max_len - 1) // 2
    off = (q_len - 1) - p
    a_cols = q_len + v_len
    nw = _NUM_CORES * _NUM_SUBCORES
    assert d == nw
    assert a_cols % _LANES == 0

    mesh = plsc.VectorSubcoreMesh(core_axis_name="c", subcore_axis_name="s")

    cp = pltpu.CompilerParams(use_tc_tiling_on_sc=False)
    if "needs_layout_passes" in pltpu.CompilerParams.__dataclass_fields__:
        cp = dataclasses.replace(cp, needs_layout_passes=False)

    @functools.partial(
        pl.kernel,
        out_type=jax.ShapeDtypeStruct((d * a_cols,), jnp.float32),
        mesh=mesh,
        compiler_params=cp,
        scratch_types=[
            pltpu.VMEM((max_len * d,), jnp.float32),
            pltpu.VMEM((a_cols,), jnp.float32),
            pltpu.SemaphoreType.DMA,
        ],
    )
    def build_at(emb_hbm, at_hbm, emb_v, row_v, sem):
        e = lax.axis_index("c") * _NUM_SUBCORES + lax.axis_index("s")
        pltpu.async_copy(emb_hbm, emb_v, sem).wait()
        lane = lax.iota(jnp.int32, _LANES)

        @pl.loop(0, a_cols, step=_LANES)
        def _(k0):
            kk = jnp.minimum(jnp.maximum((k0 - off) + lane, 0), max_len - 1)
            row_v[pl.ds(k0, _LANES)] = plsc.load_gather(emb_v, [kk * d + e])

        pltpu.async_copy(row_v, at_hbm.at[pl.ds(e * a_cols, a_cols)], sem).wait()

    return build_at


def _emit_rows_tc(q_len, v_len, max_len, d):
    a_cols = q_len + v_len
    rpb = 32
    assert q_len % rpb == 0

    sup = v_len + 128

    def body(at_ref, o_ref):
        g = pl.program_id(0)
        for r in range(rpb):
            i = g * rpb + r
            s = (q_len - 1) - i
            base = pl.multiple_of((s // 128) * 128, 128)
            t = s - base
            win = at_ref[:, pl.ds(base, sup)]
            o_ref[r] = pltpu.roll(win, sup - t, axis=1)[:, :v_len]

    return pl.pallas_call(
        body,
        grid=(q_len // rpb,),
        in_specs=[
            pl.BlockSpec((d, a_cols), lambda g: (0, 0)),
        ],
        out_specs=pl.BlockSpec((rpb, d, v_len), lambda g: (g, 0, 0)),
        out_shape=jax.ShapeDtypeStruct((q_len, d, v_len), jnp.float32),
    )


def kernel(q, v, embeddings):
    q_len = int(q.shape[1])
    v_len = int(v.shape[1])
    max_len, d = int(embeddings.shape[0]), int(embeddings.shape[1])
    at_flat = _build_at_call(q_len, v_len, max_len, d)(embeddings.reshape(max_len * d))
    a_t = at_flat.reshape(d, q_len + v_len)
    out_t = _emit_rows_tc(q_len, v_len, max_len, d)(a_t)
    return jnp.swapaxes(out_t, 1, 2)


# speedup vs baseline: 9.0176x; 1.0393x over previous
import dataclasses
import functools

import jax
import jax.numpy as jnp
from jax import lax
from jax.experimental import pallas as pl
from jax.experimental.pallas import tpu as pltpu
from jax.experimental.pallas import tpu_sc as plsc

_NUM_CORES = 2
_NUM_SUBCORES = 16
_LANES = 16


def _build_at_call(q_len, v_len, max_len, d):
    p = (max_len - 1) // 2
    off = (q_len - 1) - p
    a_cols = q_len + v_len
    nw = _NUM_CORES * _NUM_SUBCORES
    assert d == nw
    assert a_cols % _LANES == 0

    mesh = plsc.VectorSubcoreMesh(core_axis_name="c", subcore_axis_name="s")

    cp = pltpu.CompilerParams(use_tc_tiling_on_sc=False)
    if "needs_layout_passes" in pltpu.CompilerParams.__dataclass_fields__:
        cp = dataclasses.replace(cp, needs_layout_passes=False)

    @functools.partial(
        pl.kernel,
        out_type=jax.ShapeDtypeStruct((d * a_cols,), jnp.float32),
        mesh=mesh,
        compiler_params=cp,
        scratch_types=[
            pltpu.VMEM((max_len * d,), jnp.float32),
            pltpu.VMEM((a_cols,), jnp.float32),
            pltpu.SemaphoreType.DMA,
        ],
    )
    def build_at(emb_hbm, at_hbm, emb_v, row_v, sem):
        e = lax.axis_index("c") * _NUM_SUBCORES + lax.axis_index("s")
        pltpu.async_copy(emb_hbm, emb_v, sem).wait()
        lane = lax.iota(jnp.int32, _LANES)

        @pl.loop(0, a_cols, step=_LANES)
        def _(k0):
            kk = jnp.minimum(jnp.maximum((k0 - off) + lane, 0), max_len - 1)
            row_v[pl.ds(k0, _LANES)] = plsc.load_gather(emb_v, [kk * d + e])

        pltpu.async_copy(row_v, at_hbm.at[pl.ds(e * a_cols, a_cols)], sem).wait()

    return build_at


def _emit_rows_tc(q_len, v_len, max_len, d):
    a_cols = q_len + v_len
    rpb = 64
    assert q_len % rpb == 0

    sup = v_len + 128

    def body(at_ref, o_ref):
        g = pl.program_id(0)
        for r in range(rpb):
            i = g * rpb + r
            s = (q_len - 1) - i
            base = pl.multiple_of((s // 128) * 128, 128)
            t = s - base
            win = at_ref[:, pl.ds(base, sup)]
            o_ref[r] = pltpu.roll(win, sup - t, axis=1)[:, :v_len]

    return pl.pallas_call(
        body,
        grid=(q_len // rpb,),
        in_specs=[
            pl.BlockSpec((d, a_cols), lambda g: (0, 0)),
        ],
        out_specs=pl.BlockSpec((rpb, d, v_len), lambda g: (g, 0, 0)),
        out_shape=jax.ShapeDtypeStruct((q_len, d, v_len), jnp.float32),
    )


def kernel(q, v, embeddings):
    q_len = int(q.shape[1])
    v_len = int(v.shape[1])
    max_len, d = int(embeddings.shape[0]), int(embeddings.shape[1])
    at_flat = _build_at_call(q_len, v_len, max_len, d)(embeddings.reshape(max_len * d))
    a_t = at_flat.reshape(d, q_len + v_len)
    out_t = _emit_rows_tc(q_len, v_len, max_len, d)(a_t)
    return jnp.swapaxes(out_t, 1, 2)


# speedup vs baseline: 9.0354x; 1.0020x over previous
import dataclasses
import functools

import jax
import jax.numpy as jnp
from jax import lax
from jax.experimental import pallas as pl
from jax.experimental.pallas import tpu as pltpu
from jax.experimental.pallas import tpu_sc as plsc

_NUM_CORES = 2
_NUM_SUBCORES = 16
_LANES = 16


def _build_at_call(q_len, v_len, max_len, d):
    p = (max_len - 1) // 2
    off = (q_len - 1) - p
    a_cols = q_len + v_len
    nw = _NUM_CORES * _NUM_SUBCORES
    assert d == nw
    assert a_cols % _LANES == 0

    mesh = plsc.VectorSubcoreMesh(core_axis_name="c", subcore_axis_name="s")

    cp = pltpu.CompilerParams(use_tc_tiling_on_sc=False)
    if "needs_layout_passes" in pltpu.CompilerParams.__dataclass_fields__:
        cp = dataclasses.replace(cp, needs_layout_passes=False)

    @functools.partial(
        pl.kernel,
        out_type=jax.ShapeDtypeStruct((d * a_cols,), jnp.float32),
        mesh=mesh,
        compiler_params=cp,
        scratch_types=[
            pltpu.VMEM((max_len * d,), jnp.float32),
            pltpu.VMEM((a_cols,), jnp.float32),
            pltpu.SemaphoreType.DMA,
        ],
    )
    def build_at(emb_hbm, at_hbm, emb_v, row_v, sem):
        e = lax.axis_index("c") * _NUM_SUBCORES + lax.axis_index("s")
        pltpu.async_copy(emb_hbm, emb_v, sem).wait()
        lane = lax.iota(jnp.int32, _LANES)

        @pl.loop(0, a_cols, step=_LANES)
        def _(k0):
            kk = jnp.minimum(jnp.maximum((k0 - off) + lane, 0), max_len - 1)
            row_v[pl.ds(k0, _LANES)] = plsc.load_gather(emb_v, [kk * d + e])

        pltpu.async_copy(row_v, at_hbm.at[pl.ds(e * a_cols, a_cols)], sem).wait()

    return build_at


def _emit_rows_tc(q_len, v_len, max_len, d):
    a_cols = q_len + v_len
    rpb = 128
    jb = v_len // 2
    assert q_len % rpb == 0 and jb % 128 == 0

    sup = jb + 128

    def body(at_ref, o_ref):
        g = pl.program_id(0)
        h = pl.program_id(1)
        for r in range(rpb):
            i = g * rpb + r
            s = (q_len - 1) - i
            t = ((q_len - 1) - r) % rpb
            base = pl.multiple_of((s - t) + h * jb, 128)
            win = at_ref[:, pl.ds(base, sup)]
            o_ref[r] = pltpu.roll(win, sup - t, axis=1)[:, :jb]

    return pl.pallas_call(
        body,
        grid=(q_len // rpb, v_len // jb),
        in_specs=[
            pl.BlockSpec((d, a_cols), lambda g, h: (0, 0)),
        ],
        out_specs=pl.BlockSpec((rpb, d, jb), lambda g, h: (g, 0, h)),
        out_shape=jax.ShapeDtypeStruct((q_len, d, v_len), jnp.float32),
    )


def kernel(q, v, embeddings):
    q_len = int(q.shape[1])
    v_len = int(v.shape[1])
    max_len, d = int(embeddings.shape[0]), int(embeddings.shape[1])
    at_flat = _build_at_call(q_len, v_len, max_len, d)(embeddings.reshape(max_len * d))
    a_t = at_flat.reshape(d, q_len + v_len)
    out_t = _emit_rows_tc(q_len, v_len, max_len, d)(a_t)
    return jnp.swapaxes(out_t, 1, 2)


# speedup vs baseline: 9.0539x; 1.0020x over previous
import dataclasses
import functools

import jax
import jax.numpy as jnp
from jax import lax
from jax.experimental import pallas as pl
from jax.experimental.pallas import tpu as pltpu
from jax.experimental.pallas import tpu_sc as plsc

_NUM_CORES = 2
_NUM_SUBCORES = 16
_LANES = 16


def _build_at_call(q_len, v_len, max_len, d):
    p = (max_len - 1) // 2
    off = (q_len - 1) - p
    a_cols = q_len + v_len
    nw = _NUM_CORES * _NUM_SUBCORES
    assert d == nw
    assert a_cols % _LANES == 0

    mesh = plsc.VectorSubcoreMesh(core_axis_name="c", subcore_axis_name="s")

    cp = pltpu.CompilerParams(use_tc_tiling_on_sc=False)
    if "needs_layout_passes" in getattr(pltpu.CompilerParams, "__dataclass_fields__", {}):
        cp = dataclasses.replace(cp, needs_layout_passes=False)

    @functools.partial(
        pl.kernel,
        out_type=jax.ShapeDtypeStruct((d * a_cols,), jnp.float32),
        mesh=mesh,
        compiler_params=cp,
        scratch_types=[
            pltpu.VMEM((max_len * d,), jnp.float32),
            pltpu.VMEM((a_cols,), jnp.float32),
            pltpu.SemaphoreType.DMA,
        ],
    )
    def build_at(emb_hbm, at_hbm, emb_v, row_v, sem):
        e = lax.axis_index("c") * _NUM_SUBCORES + lax.axis_index("s")
        pltpu.async_copy(emb_hbm, emb_v, sem).wait()
        lane = lax.iota(jnp.int32, _LANES)

        @pl.loop(0, a_cols, step=_LANES)
        def _(k0):
            kk = jnp.minimum(jnp.maximum((k0 - off) + lane, 0), max_len - 1)
            row_v[pl.ds(k0, _LANES)] = plsc.load_gather(emb_v, [kk * d + e])

        pltpu.async_copy(row_v, at_hbm.at[pl.ds(e * a_cols, a_cols)], sem).wait()

    return build_at


def _emit_rows_tc(q_len, v_len, max_len, d):
    a_cols = q_len + v_len
    rpb = 128
    jb = v_len // 2
    assert q_len % rpb == 0 and jb % 128 == 0 and rpb % 128 == 0

    sup = jb + 128

    def body(at_ref, o_ref):
        g = pl.program_id(0)
        h = pl.program_id(1)
        for r in range(rpb):
            i = g * rpb + r
            s = (q_len - 1) - i
            t = ((q_len - 1) - r) % 128
            base = pl.multiple_of((s - t) + h * jb, 128)
            win = at_ref[:, pl.ds(base, sup)]
            o_ref[r] = pltpu.roll(win, sup - t, axis=1)[:, :jb]

    return pl.pallas_call(
        body,
        grid=(q_len // rpb, v_len // jb),
        in_specs=[
            pl.BlockSpec((d, a_cols), lambda g, h: (0, 0)),
        ],
        out_specs=pl.BlockSpec((rpb, d, jb), lambda g, h: (g, 0, h)),
        out_shape=jax.ShapeDtypeStruct((q_len, d, v_len), jnp.float32),
    )


def kernel(q, v, embeddings):
    q_len = int(q.shape[1])
    v_len = int(v.shape[1])
    max_len, d = int(embeddings.shape[0]), int(embeddings.shape[1])
    at_flat = _build_at_call(q_len, v_len, max_len, d)(embeddings.reshape(max_len * d))
    a_t = at_flat.reshape(d, q_len + v_len)
    out_t = _emit_rows_tc(q_len, v_len, max_len, d)(a_t)
    return jnp.swapaxes(out_t, 1, 2)
